# baseline (speedup 1.0000x reference)
import jax
import jax.numpy as jnp
from jax.experimental import pallas as pl

B, N, H = 16384, 1024, 256
TILE_B = 1024


def _retrieval_kernel(z_ref, mem_ref, out_ref):
    z = z_ref[...]
    mem = mem_ref[...]

    z_norm = z * jax.lax.rsqrt(jnp.maximum(jnp.sum(z * z, axis=1, keepdims=True), 1e-24))

    m_inv = jax.lax.rsqrt(jnp.maximum(jnp.sum(mem * mem, axis=1), 1e-24))

    mem_bf = mem.astype(jnp.bfloat16)
    sim = jax.lax.dot_general(
        z_norm.astype(jnp.bfloat16), mem_bf,
        (((1,), (1,)), ((), ())),
        preferred_element_type=jnp.float32,
    )
    sim = sim * m_inv[None, :]

    sim_max = jnp.max(sim, axis=1, keepdims=True)
    e = jnp.exp(sim - sim_max)
    w = e / jnp.sum(e, axis=1, keepdims=True)

    out_ref[...] = jnp.dot(w.astype(jnp.bfloat16), mem_bf,
                           preferred_element_type=jnp.float32)


def kernel(z, memory):
    return pl.pallas_call(
        _retrieval_kernel,
        grid=(B // TILE_B,),
        in_specs=[
            pl.BlockSpec((TILE_B, H), lambda i: (i, 0)),
            pl.BlockSpec((N, H), lambda i: (0, 0)),
        ],
        out_specs=pl.BlockSpec((TILE_B, H), lambda i: (i, 0)),
        out_shape=jax.ShapeDtypeStruct((B, H), jnp.float32),
    )(z, memory)


# speedup vs baseline: 1.7687x; 1.7687x over previous
import jax
import jax.numpy as jnp
from jax.experimental import pallas as pl

B, N, H = 16384, 1024, 256
TILE_B = 1024


def _retrieval_kernel(z_ref, mem_ref, out_ref):
    z = z_ref[...]
    mem = mem_ref[...]

    z_norm = z * jax.lax.rsqrt(jnp.maximum(jnp.sum(z * z, axis=1, keepdims=True), 1e-24))

    m_inv = jax.lax.rsqrt(jnp.maximum(jnp.sum(mem * mem, axis=1), 1e-24))

    mem_norm_bf = (mem * m_inv[:, None]).astype(jnp.bfloat16)
    sim = jax.lax.dot_general(
        z_norm.astype(jnp.bfloat16), mem_norm_bf,
        (((1,), (1,)), ((), ())),
        preferred_element_type=jnp.float32,
    )

    e = jnp.exp(sim)
    inv_sum = 1.0 / jnp.sum(e, axis=1, keepdims=True)

    acc = jnp.dot(e.astype(jnp.bfloat16), mem.astype(jnp.bfloat16),
                  preferred_element_type=jnp.float32)
    out_ref[...] = acc * inv_sum


def kernel(z, memory):
    return pl.pallas_call(
        _retrieval_kernel,
        grid=(B // TILE_B,),
        in_specs=[
            pl.BlockSpec((TILE_B, H), lambda i: (i, 0)),
            pl.BlockSpec((N, H), lambda i: (0, 0)),
        ],
        out_specs=pl.BlockSpec((TILE_B, H), lambda i: (i, 0)),
        out_shape=jax.ShapeDtypeStruct((B, H), jnp.float32),
    )(z, memory)
